# baseline (device time: 107779 ns/iter reference)
import jax
import jax.numpy as jnp
from jax import lax
from jax.experimental import pallas as pl
from jax.experimental.pallas import tpu as pltpu

N_DEV = 32


def _gelu(y):
    c = 0.7978845608028654
    return 0.5 * y * (1.0 + jnp.tanh(c * (y + 0.044715 * y * y * y)))


def kernel(x, w_mat):
    m, k_per = x.shape
    _, n = w_mat.shape
    m_per = m // N_DEV

    def body(x_ref, w_ref, out_ref, part_ref, recv_ref, send_sems, recv_sems):
        my = lax.axis_index("i")
        left = (my - 1) % N_DEV
        right = (my + 1) % N_DEV

        barrier_sem = pltpu.get_barrier_semaphore()
        for nbr in (left, right):
            pl.semaphore_signal(
                barrier_sem, inc=1,
                device_id=(nbr,), device_id_type=pl.DeviceIdType.MESH,
            )
        pl.semaphore_wait(barrier_sem, 2)

        part_ref[...] = jnp.dot(
            x_ref[...], w_ref[...], preferred_element_type=jnp.float32
        )

        rdmas = []
        c0 = (my - 1) % N_DEV
        rdma = pltpu.make_async_remote_copy(
            src_ref=part_ref.at[pl.ds(c0 * m_per, m_per), :],
            dst_ref=recv_ref.at[0],
            send_sem=send_sems.at[0],
            recv_sem=recv_sems.at[0],
            device_id=(right,),
            device_id_type=pl.DeviceIdType.MESH,
        )
        rdma.start()
        rdma.wait()

        for s in range(1, N_DEV - 1):
            c = (my - s - 1) % N_DEV
            recv_ref[s - 1, :, :] = (
                recv_ref[s - 1, :, :] + part_ref[pl.ds(c * m_per, m_per), :]
            )
            rdma = pltpu.make_async_remote_copy(
                src_ref=recv_ref.at[s - 1],
                dst_ref=recv_ref.at[s],
                send_sem=send_sems.at[s],
                recv_sem=recv_sems.at[s],
                device_id=(right,),
                device_id_type=pl.DeviceIdType.MESH,
            )
            rdma.start()
            rdma.wait()

        final = (
            recv_ref[N_DEV - 2, :, :]
            + part_ref[pl.ds(my * m_per, m_per), :]
        )
        out_ref[...] = _gelu(final)

    return pl.pallas_call(
        body,
        out_shape=jax.ShapeDtypeStruct((m_per, n), jnp.float32),
        in_specs=[
            pl.BlockSpec(memory_space=pltpu.VMEM),
            pl.BlockSpec(memory_space=pltpu.VMEM),
        ],
        out_specs=pl.BlockSpec(memory_space=pltpu.VMEM),
        scratch_shapes=[
            pltpu.VMEM((m, n), jnp.float32),
            pltpu.VMEM((N_DEV - 1, m_per, n), jnp.float32),
            pltpu.SemaphoreType.DMA((N_DEV - 1,)),
            pltpu.SemaphoreType.DMA((N_DEV - 1,)),
        ],
        compiler_params=pltpu.CompilerParams(collective_id=0),
    )(x, w_mat)


# device time: 45307 ns/iter; 2.3789x vs baseline; 2.3789x over previous
import jax
import jax.numpy as jnp
from jax import lax
from jax.experimental import pallas as pl
from jax.experimental.pallas import tpu as pltpu

N_DEV = 32
H = 512


def _gelu(v):
    c = 0.7978845608028654
    return 0.5 * v * (1.0 + jnp.tanh(c * (v + 0.044715 * v * v * v)))


def kernel(x, w_mat):
    m, k_per = x.shape
    _, n = w_mat.shape
    m_per = m // N_DEV

    def body(x_ref, w_ref, out_ref, part, rA, rB, rA2, rB2, rX,
             szA, rzA, syB, ryB, syA, ryA, szB, rzB, sX, rXs):
        p = lax.axis_index("i")
        z = p // 8
        j = p % 8
        y = j // 2
        xc = (j + y) % 2
        t = p % 2
        px = p + 1 - 2 * t

        right_z = (p + 8) % N_DEV
        left_z = (p - 8) % N_DEV
        yn = (y + 1) % 4
        yp = (y - 1) % 4
        right_y = 8 * z + 2 * yn + (xc + yn) % 2
        left_y = 8 * z + 2 * yp + (xc + yp) % 2

        barrier_sem = pltpu.get_barrier_semaphore()
        for nbr in (right_z, left_z, right_y, left_y, px):
            pl.semaphore_signal(
                barrier_sem, inc=1,
                device_id=(nbr,), device_id_type=pl.DeviceIdType.MESH,
            )
        pl.semaphore_wait(barrier_sem, 5)

        part[...] = jnp.dot(
            x_ref[...], w_ref[...], preferred_element_type=jnp.float32
        )

        colsA = pl.ds(0, H)
        colsB = pl.ds(H, H)

        def rdma(src, dst, ssem, rsem, tgt):
            r = pltpu.make_async_remote_copy(
                src_ref=src, dst_ref=dst, send_sem=ssem, recv_sem=rsem,
                device_id=(tgt,), device_id_type=pl.DeviceIdType.MESH,
            )
            r.start()
            return r

        cA = [(z - s - 1) % 4 for s in range(3)] + [z]
        cB = [(y - s - 1) % 4 for s in range(3)] + [y]

        a0 = rdma(part.at[pl.ds(cA[0] * 256, 256), colsA], rA.at[0],
                  szA.at[0], rzA.at[0], right_z)
        b0 = [rdma(part.at[pl.ds(256 * k + 64 * cB[0], 64), colsB],
                   rB.at[0, pl.ds(64 * k, 64), :],
                   syB.at[0 * 4 + k], ryB.at[0 * 4 + k], right_y)
              for k in range(4)]

        a_prev = a0
        b_prev = b0
        for s in (1, 2):
            a_prev.wait()
            rA[s - 1, :, :] = (
                rA[s - 1, :, :] + part[pl.ds(cA[s] * 256, 256), colsA]
            )
            a_prev = rdma(rA.at[s - 1], rA.at[s], szA.at[s], rzA.at[s],
                          right_z)
            for k in range(4):
                b_prev[k].wait()
            for k in range(4):
                rB[s - 1, pl.ds(64 * k, 64), :] = (
                    rB[s - 1, pl.ds(64 * k, 64), :]
                    + part[pl.ds(256 * k + 64 * cB[s], 64), colsB]
                )
            b_prev = [rdma(rB.at[s - 1, pl.ds(64 * k, 64), :],
                           rB.at[s, pl.ds(64 * k, 64), :],
                           syB.at[s * 4 + k], ryB.at[s * 4 + k], right_y)
                      for k in range(4)]

        a_prev.wait()
        rA[2, :, :] = rA[2, :, :] + part[pl.ds(cA[3] * 256, 256), colsA]
        for k in range(4):
            b_prev[k].wait()
        for k in range(4):
            rB[2, pl.ds(64 * k, 64), :] = (
                rB[2, pl.ds(64 * k, 64), :]
                + part[pl.ds(256 * k + 64 * cB[3], 64), colsB]
            )

        cA2 = [(y - s - 1) % 4 for s in range(3)] + [y]
        cB2 = [(z - s - 1) % 4 for s in range(3)] + [z]

        a0 = rdma(rA.at[2, pl.ds(64 * cA2[0], 64), :], rA2.at[0],
                  syA.at[0], ryA.at[0], right_y)
        b0 = rdma(rB.at[2, pl.ds(64 * cB2[0], 64), :], rB2.at[0],
                  szB.at[0], rzB.at[0], right_z)
        a_prev, b_prev = a0, b0
        for s in (1, 2):
            a_prev.wait()
            rA2[s - 1, :, :] = (
                rA2[s - 1, :, :] + rA[2, pl.ds(64 * cA2[s], 64), :]
            )
            a_prev = rdma(rA2.at[s - 1], rA2.at[s], syA.at[s], ryA.at[s],
                          right_y)
            b_prev.wait()
            rB2[s - 1, :, :] = (
                rB2[s - 1, :, :] + rB[2, pl.ds(64 * cB2[s], 64), :]
            )
            b_prev = rdma(rB2.at[s - 1], rB2.at[s], szB.at[s], rzB.at[s],
                          right_z)
        a_prev.wait()
        rA2[2, :, :] = rA2[2, :, :] + rA[2, pl.ds(64 * cA2[3], 64), :]
        b_prev.wait()
        rB2[2, :, :] = rB2[2, :, :] + rB[2, pl.ds(64 * cB2[3], 64), :]

        u = 1 - t
        xa = rdma(rA2.at[2, pl.ds(32 * u, 32), :], rX.at[0],
                  sX.at[0], rXs.at[0], px)
        xb = rdma(rB2.at[2, pl.ds(32 * u, 32), :], rX.at[1],
                  sX.at[1], rXs.at[1], px)
        xa.wait()
        xb.wait()
        out_ref[:, colsA] = _gelu(rA2[2, pl.ds(32 * t, 32), :] + rX[0, :, :])
        out_ref[:, colsB] = _gelu(rB2[2, pl.ds(32 * t, 32), :] + rX[1, :, :])

    return pl.pallas_call(
        body,
        out_shape=jax.ShapeDtypeStruct((m_per, n), jnp.float32),
        in_specs=[
            pl.BlockSpec(memory_space=pltpu.VMEM),
            pl.BlockSpec(memory_space=pltpu.VMEM),
        ],
        out_specs=pl.BlockSpec(memory_space=pltpu.VMEM),
        scratch_shapes=[
            pltpu.VMEM((m, n), jnp.float32),
            pltpu.VMEM((3, 256, H), jnp.float32),
            pltpu.VMEM((3, 256, H), jnp.float32),
            pltpu.VMEM((3, 64, H), jnp.float32),
            pltpu.VMEM((3, 64, H), jnp.float32),
            pltpu.VMEM((2, 32, H), jnp.float32),
            pltpu.SemaphoreType.DMA((3,)),
            pltpu.SemaphoreType.DMA((3,)),
            pltpu.SemaphoreType.DMA((12,)),
            pltpu.SemaphoreType.DMA((12,)),
            pltpu.SemaphoreType.DMA((3,)),
            pltpu.SemaphoreType.DMA((3,)),
            pltpu.SemaphoreType.DMA((3,)),
            pltpu.SemaphoreType.DMA((3,)),
            pltpu.SemaphoreType.DMA((2,)),
            pltpu.SemaphoreType.DMA((2,)),
        ],
        compiler_params=pltpu.CompilerParams(collective_id=0),
    )(x, w_mat)


# device time: 41933 ns/iter; 2.5703x vs baseline; 1.0805x over previous
import jax
import jax.numpy as jnp
from jax import lax
from jax.experimental import pallas as pl
from jax.experimental.pallas import tpu as pltpu

N_DEV = 32
H = 512


def _gelu(v):
    c = 0.7978845608028654
    return 0.5 * v * (1.0 + jnp.tanh(c * (v + 0.044715 * v * v * v)))


def kernel(x, w_mat):
    m, k_per = x.shape
    _, n = w_mat.shape
    m_per = m // N_DEV

    def body(x_ref, w_ref, out_ref, part, rA, rB, rA2, rB2, rX,
             szA, rzA, syB, ryB, syA, ryA, szB, rzB, sX, rXs):
        p = lax.axis_index("i")
        z = p // 8
        j = p % 8
        y = j // 2
        xc = (j + y) % 2
        t = p % 2
        px = p + 1 - 2 * t

        right_z = (p + 8) % N_DEV
        left_z = (p - 8) % N_DEV
        yn = (y + 1) % 4
        yp = (y - 1) % 4
        right_y = 8 * z + 2 * yn + (xc + yn) % 2
        left_y = 8 * z + 2 * yp + (xc + yp) % 2

        barrier_sem = pltpu.get_barrier_semaphore()
        for nbr in (right_z, left_z, right_y, left_y, px):
            pl.semaphore_signal(
                barrier_sem, inc=1,
                device_id=(nbr,), device_id_type=pl.DeviceIdType.MESH,
            )
        pl.semaphore_wait(barrier_sem, 5)

        colsA = pl.ds(0, H)
        colsB = pl.ds(H, H)

        def rdma(src, dst, ssem, rsem, tgt):
            r = pltpu.make_async_remote_copy(
                src_ref=src, dst_ref=dst, send_sem=ssem, recv_sem=rsem,
                device_id=(tgt,), device_id_type=pl.DeviceIdType.MESH,
            )
            r.start()
            return r

        cA = [(z - s - 1) % 4 for s in range(3)] + [z]
        cB = [(y - s - 1) % 4 for s in range(3)] + [y]

        part[:, colsA] = jnp.dot(
            x_ref[...], w_ref[:, colsA], preferred_element_type=jnp.float32
        )
        qs = [pl.ds(256 * q, 256) for q in range(2)]
        aC = [rdma(part.at[pl.ds(cA[0] * 256, 256), qs[q]],
                   rA.at[0, :, qs[q]], szA.at[q], rzA.at[q], right_z)
              for q in range(2)]
        part[:, colsB] = jnp.dot(
            x_ref[...], w_ref[:, colsB], preferred_element_type=jnp.float32
        )
        bC = [rdma(part.at[pl.ds(256 * k + 64 * cB[0], 64), colsB],
                   rB.at[0, pl.ds(64 * k, 64), :],
                   syB.at[k], ryB.at[k], right_y)
              for k in range(4)]

        for s in (1, 2):
            na = []
            for q in range(2):
                aC[q].wait()
                rA[s - 1, :, qs[q]] = (
                    rA[s - 1, :, qs[q]]
                    + part[pl.ds(cA[s] * 256, 256), qs[q]]
                )
                na.append(rdma(rA.at[s - 1, :, qs[q]], rA.at[s, :, qs[q]],
                               szA.at[s * 2 + q], rzA.at[s * 2 + q],
                               right_z))
            nb = []
            for k in range(4):
                bC[k].wait()
                rB[s - 1, pl.ds(64 * k, 64), :] = (
                    rB[s - 1, pl.ds(64 * k, 64), :]
                    + part[pl.ds(256 * k + 64 * cB[s], 64), colsB]
                )
                nb.append(rdma(rB.at[s - 1, pl.ds(64 * k, 64), :],
                               rB.at[s, pl.ds(64 * k, 64), :],
                               syB.at[s * 4 + k], ryB.at[s * 4 + k],
                               right_y))
            aC, bC = na, nb

        cA2 = [(y - s - 1) % 4 for s in range(3)] + [y]
        cB2 = [(z - s - 1) % 4 for s in range(3)] + [z]

        for q in range(2):
            aC[q].wait()
            rA[2, :, qs[q]] = (
                rA[2, :, qs[q]] + part[pl.ds(cA[3] * 256, 256), qs[q]]
            )
        a2C = [rdma(rA.at[2, pl.ds(64 * cA2[0], 64), qs[q]],
                    rA2.at[0, :, qs[q]], syA.at[q], ryA.at[q], right_y)
               for q in range(2)]

        for k in range(4):
            bC[k].wait()
            rB[2, pl.ds(64 * k, 64), :] = (
                rB[2, pl.ds(64 * k, 64), :]
                + part[pl.ds(256 * k + 64 * cB[3], 64), colsB]
            )
        b2C = [rdma(rB.at[2, pl.ds(64 * cB2[0], 64), qs[q]],
                    rB2.at[0, :, qs[q]], szB.at[q], rzB.at[q], right_z)
               for q in range(2)]

        for s in (1, 2):
            na = []
            for q in range(2):
                a2C[q].wait()
                rA2[s - 1, :, qs[q]] = (
                    rA2[s - 1, :, qs[q]]
                    + rA[2, pl.ds(64 * cA2[s], 64), qs[q]]
                )
                na.append(rdma(rA2.at[s - 1, :, qs[q]], rA2.at[s, :, qs[q]],
                               syA.at[s * 2 + q], ryA.at[s * 2 + q],
                               right_y))
            nb = []
            for q in range(2):
                b2C[q].wait()
                rB2[s - 1, :, qs[q]] = (
                    rB2[s - 1, :, qs[q]]
                    + rB[2, pl.ds(64 * cB2[s], 64), qs[q]]
                )
                nb.append(rdma(rB2.at[s - 1, :, qs[q]], rB2.at[s, :, qs[q]],
                               szB.at[s * 2 + q], rzB.at[s * 2 + q],
                               right_z))
            a2C, b2C = na, nb

        u = 1 - t
        for q in range(2):
            a2C[q].wait()
            rA2[2, :, qs[q]] = (
                rA2[2, :, qs[q]] + rA[2, pl.ds(64 * cA2[3], 64), qs[q]]
            )
        xa = rdma(rA2.at[2, pl.ds(32 * u, 32), :], rX.at[0],
                  sX.at[0], rXs.at[0], px)
        for q in range(2):
            b2C[q].wait()
            rB2[2, :, qs[q]] = (
                rB2[2, :, qs[q]] + rB[2, pl.ds(64 * cB2[3], 64), qs[q]]
            )
        xb = rdma(rB2.at[2, pl.ds(32 * u, 32), :], rX.at[1],
                  sX.at[1], rXs.at[1], px)
        xa.wait()
        out_ref[:, colsA] = _gelu(rA2[2, pl.ds(32 * t, 32), :] + rX[0, :, :])
        xb.wait()
        out_ref[:, colsB] = _gelu(rB2[2, pl.ds(32 * t, 32), :] + rX[1, :, :])

    return pl.pallas_call(
        body,
        out_shape=jax.ShapeDtypeStruct((m_per, n), jnp.float32),
        in_specs=[
            pl.BlockSpec(memory_space=pltpu.VMEM),
            pl.BlockSpec(memory_space=pltpu.VMEM),
        ],
        out_specs=pl.BlockSpec(memory_space=pltpu.VMEM),
        scratch_shapes=[
            pltpu.VMEM((m, n), jnp.float32),
            pltpu.VMEM((3, 256, H), jnp.float32),
            pltpu.VMEM((3, 256, H), jnp.float32),
            pltpu.VMEM((3, 64, H), jnp.float32),
            pltpu.VMEM((3, 64, H), jnp.float32),
            pltpu.VMEM((2, 32, H), jnp.float32),
            pltpu.SemaphoreType.DMA((6,)),
            pltpu.SemaphoreType.DMA((6,)),
            pltpu.SemaphoreType.DMA((12,)),
            pltpu.SemaphoreType.DMA((12,)),
            pltpu.SemaphoreType.DMA((6,)),
            pltpu.SemaphoreType.DMA((6,)),
            pltpu.SemaphoreType.DMA((6,)),
            pltpu.SemaphoreType.DMA((6,)),
            pltpu.SemaphoreType.DMA((2,)),
            pltpu.SemaphoreType.DMA((2,)),
        ],
        compiler_params=pltpu.CompilerParams(collective_id=0),
    )(x, w_mat)


# device time: 37925 ns/iter; 2.8419x vs baseline; 1.1057x over previous
import jax
import jax.numpy as jnp
from jax import lax
from jax.experimental import pallas as pl
from jax.experimental.pallas import tpu as pltpu

N_DEV = 32
H = 512


def _gelu(v):
    c = 0.7978845608028654
    return 0.5 * v * (1.0 + jnp.tanh(c * (v + 0.044715 * v * v * v)))


def kernel(x, w_mat):
    m, k_per = x.shape
    _, n = w_mat.shape
    m_per = m // N_DEV

    def body(x_ref, w_ref, out_ref, part, rA, rB, rA2, rB2, rX,
             szA, rzA, syB, ryB, syA, ryA, szB, rzB, sX, rXs):
        p = lax.axis_index("i")
        z = p // 8
        j = p % 8
        y = j // 2
        xc = (j + y) % 2
        t = p % 2
        px = p + 1 - 2 * t

        right_z = (p + 8) % N_DEV
        left_z = (p - 8) % N_DEV
        yn = (y + 1) % 4
        yp = (y - 1) % 4
        right_y = 8 * z + 2 * yn + (xc + yn) % 2
        left_y = 8 * z + 2 * yp + (xc + yp) % 2

        colsA = pl.ds(0, H)
        colsB = pl.ds(H, H)
        qs = [pl.ds(256 * q, 256) for q in range(2)]

        def rdma(src, dst, ssem, rsem, tgt):
            r = pltpu.make_async_remote_copy(
                src_ref=src, dst_ref=dst, send_sem=ssem, recv_sem=rsem,
                device_id=(tgt,), device_id_type=pl.DeviceIdType.MESH,
            )
            r.start()
            return r

        cA = [(z - s - 1) % 4 for s in range(3)] + [z]
        cB = [(y - s - 1) % 4 for s in range(3)] + [y]
        cA2 = [(y - s - 1) % 4 for s in range(3)] + [y]
        kB = [(z - 1 - i) % 4 for i in range(4)]

        part[:, colsA] = jnp.dot(
            x_ref[...], w_ref[:, colsA], preferred_element_type=jnp.float32
        )
        barrier_sem = pltpu.get_barrier_semaphore()
        for nbr in (right_z, left_z, right_y, left_y, px):
            pl.semaphore_signal(
                barrier_sem, inc=1,
                device_id=(nbr,), device_id_type=pl.DeviceIdType.MESH,
            )
        pl.semaphore_wait(barrier_sem, 5)

        aC = [rdma(part.at[pl.ds(cA[0] * 256, 256), qs[q]],
                   rA.at[0, :, qs[q]], szA.at[q], rzA.at[q], right_z)
              for q in range(2)]
        part[:, colsB] = jnp.dot(
            x_ref[...], w_ref[:, colsB], preferred_element_type=jnp.float32
        )
        bC = [rdma(part.at[pl.ds(256 * kB[i] + 64 * cB[0], 64), colsB],
                   rB.at[0, pl.ds(64 * i, 64), :],
                   syB.at[i], ryB.at[i], right_y)
              for i in range(4)]

        for s in (1, 2):
            na = []
            for q in range(2):
                aC[q].wait()
                rA[s - 1, :, qs[q]] = (
                    rA[s - 1, :, qs[q]]
                    + part[pl.ds(cA[s] * 256, 256), qs[q]]
                )
                na.append(rdma(rA.at[s - 1, :, qs[q]], rA.at[s, :, qs[q]],
                               szA.at[s * 2 + q], rzA.at[s * 2 + q],
                               right_z))
            nb = []
            for i in range(4):
                bC[i].wait()
                rB[s - 1, pl.ds(64 * i, 64), :] = (
                    rB[s - 1, pl.ds(64 * i, 64), :]
                    + part[pl.ds(256 * kB[i] + 64 * cB[s], 64), colsB]
                )
                nb.append(rdma(rB.at[s - 1, pl.ds(64 * i, 64), :],
                               rB.at[s, pl.ds(64 * i, 64), :],
                               syB.at[s * 4 + i], ryB.at[s * 4 + i],
                               right_y))
            aC, bC = na, nb

        a2C = [None, None]
        for q in range(2):
            aC[q].wait()
            rA[2, :, qs[q]] = (
                rA[2, :, qs[q]] + part[pl.ds(cA[3] * 256, 256), qs[q]]
            )
            a2C[q] = rdma(rA.at[2, pl.ds(64 * cA2[0], 64), qs[q]],
                          rA2.at[0, :, qs[q]], syA.at[q], ryA.at[q],
                          right_y)

        b2C = [None, None]
        for i in range(4):
            bC[i].wait()
            rB[2, pl.ds(64 * i, 64), :] = (
                rB[2, pl.ds(64 * i, 64), :]
                + part[pl.ds(256 * kB[i] + 64 * cB[3], 64), colsB]
            )
            if i == 0:
                for q in range(2):
                    b2C[q] = rdma(rB.at[2, pl.ds(0, 64), qs[q]],
                                  rB2.at[0, :, qs[q]], szB.at[q],
                                  rzB.at[q], right_z)

        for s in (1, 2):
            na = [None, None]
            for q in range(2):
                a2C[q].wait()
                rA2[s - 1, :, qs[q]] = (
                    rA2[s - 1, :, qs[q]]
                    + rA[2, pl.ds(64 * cA2[s], 64), qs[q]]
                )
                na[q] = rdma(rA2.at[s - 1, :, qs[q]], rA2.at[s, :, qs[q]],
                             syA.at[s * 2 + q], ryA.at[s * 2 + q], right_y)
            nb = [None, None]
            for q in range(2):
                b2C[q].wait()
                rB2[s - 1, :, qs[q]] = (
                    rB2[s - 1, :, qs[q]]
                    + rB[2, pl.ds(64 * s, 64), qs[q]]
                )
                nb[q] = rdma(rB2.at[s - 1, :, qs[q]], rB2.at[s, :, qs[q]],
                             szB.at[s * 2 + q], rzB.at[s * 2 + q], right_z)
            a2C, b2C = na, nb

        u = 1 - t
        xA = [None, None]
        for q in range(2):
            a2C[q].wait()
            rA2[2, :, qs[q]] = (
                rA2[2, :, qs[q]] + rA[2, pl.ds(64 * cA2[3], 64), qs[q]]
            )
            xA[q] = rdma(rA2.at[2, pl.ds(32 * u, 32), qs[q]],
                         rX.at[0, :, qs[q]], sX.at[q], rXs.at[q], px)
        xB = [None, None]
        for q in range(2):
            b2C[q].wait()
            rB2[2, :, qs[q]] = (
                rB2[2, :, qs[q]] + rB[2, pl.ds(64 * 3, 64), qs[q]]
            )
            xB[q] = rdma(rB2.at[2, pl.ds(32 * u, 32), qs[q]],
                         rX.at[1, :, qs[q]], sX.at[2 + q], rXs.at[2 + q],
                         px)
        for q in range(2):
            xA[q].wait()
            out_ref[:, qs[q]] = _gelu(
                rA2[2, pl.ds(32 * t, 32), qs[q]] + rX[0, :, qs[q]]
            )
        for q in range(2):
            xB[q].wait()
            out_ref[:, pl.ds(H + 256 * q, 256)] = _gelu(
                rB2[2, pl.ds(32 * t, 32), qs[q]] + rX[1, :, qs[q]]
            )

    return pl.pallas_call(
        body,
        out_shape=jax.ShapeDtypeStruct((m_per, n), jnp.float32),
        in_specs=[
            pl.BlockSpec(memory_space=pltpu.VMEM),
            pl.BlockSpec(memory_space=pltpu.VMEM),
        ],
        out_specs=pl.BlockSpec(memory_space=pltpu.VMEM),
        scratch_shapes=[
            pltpu.VMEM((m, n), jnp.float32),
            pltpu.VMEM((3, 256, H), jnp.float32),
            pltpu.VMEM((3, 256, H), jnp.float32),
            pltpu.VMEM((3, 64, H), jnp.float32),
            pltpu.VMEM((3, 64, H), jnp.float32),
            pltpu.VMEM((2, 32, H), jnp.float32),
            pltpu.SemaphoreType.DMA((6,)),
            pltpu.SemaphoreType.DMA((6,)),
            pltpu.SemaphoreType.DMA((12,)),
            pltpu.SemaphoreType.DMA((12,)),
            pltpu.SemaphoreType.DMA((6,)),
            pltpu.SemaphoreType.DMA((6,)),
            pltpu.SemaphoreType.DMA((6,)),
            pltpu.SemaphoreType.DMA((6,)),
            pltpu.SemaphoreType.DMA((4,)),
            pltpu.SemaphoreType.DMA((4,)),
        ],
        compiler_params=pltpu.CompilerParams(collective_id=0),
    )(x, w_mat)
